# baseline (speedup 1.0000x reference)
import functools

import jax
import jax.numpy as jnp
from jax import lax
from jax.experimental import pallas as pl
from jax.experimental.pallas import tpu as pltpu
from jax.experimental.pallas import tpu_sc as plsc

N = 10000
E = 320000
D = 128

NC = 2
NS = 16
NW = NC * NS
EPW = E // NW
K = 80
NCHUNK = EPW // K
NBUF = 5

@functools.lru_cache(maxsize=None)
def _get_mesh():
    return plsc.VectorSubcoreMesh(core_axis_name="c", subcore_axis_name="s",
                                  num_cores=NC, num_subcores=NS)


def _repack_idx(idx1, idx2):
    def rep(i, _):
        for l in range(K // 16):
            idx2[i, pl.ds(l * 16, 16)] = idx1[pl.ds(i * K + l * 16, 16)]
        return ()

    lax.fori_loop(0, NCHUNK, rep, (), unroll=False)


def _deg_body(dst_hbm, out_hbm, didx1, didx2, ones_v, zbuf, dacc, sem, semi):
    c = lax.axis_index("c")
    s = lax.axis_index("s")
    wid = c * NS + s
    idma = pltpu.async_copy(dst_hbm.at[pl.ds(wid * EPW, EPW)], didx1, semi)
    for j in range(2000 // 16):
        zbuf[pl.ds(j * 16, 16)] = jnp.zeros((16,), jnp.float32)
    for j in range(K // 16):
        ones_v[pl.ds(j * 16, 16)] = jnp.full((16,), 1.0, jnp.float32)

    @pl.when(s < 5)
    def _():
        pltpu.sync_copy(zbuf, dacc.at[pl.ds(s * 2000, 2000)])
    idma.wait()
    _repack_idx(didx1, didx2)
    plsc.subcore_barrier()

    def block(i0):
        sds = [pltpu.async_copy(ones_v, dacc.at[didx2.at[i0 + b]], sem,
                                add=True)
               for b in range(NBUF)]
        for sd in sds:
            sd.wait()

    pl.loop(0, NCHUNK, step=NBUF)(block)
    plsc.subcore_barrier()

    @pl.when(s < 5)
    def _():
        pltpu.sync_copy(dacc.at[pl.ds(s * 2000, 2000)], zbuf)
        pltpu.sync_copy(zbuf, out_hbm.at[pl.ds(c * N + s * 2000, 2000)])


@functools.lru_cache(maxsize=None)
def _deg_kernel():
    return pl.kernel(
        _deg_body,
        out_type=jax.ShapeDtypeStruct((NC * N,), jnp.float32),
        mesh=_get_mesh(),
        scratch_types=[
            pltpu.VMEM((EPW,), jnp.int32),
            pltpu.VMEM((NCHUNK, K), jnp.int32),
            pltpu.VMEM((K,), jnp.float32),
            pltpu.VMEM((2000,), jnp.float32),
            pltpu.VMEM_SHARED((N,), jnp.float32),
            pltpu.SemaphoreType.DMA,
            pltpu.SemaphoreType.DMA,
        ],
        compiler_params=pltpu.CompilerParams(use_tc_tiling_on_sc=False),
    )


DH = D // 2


def _mp_body(src_hbm, dst_hbm, hs_hbm, out_hbm,
             sidx1, didx1, didx2, rows, zr, acc,
             sg0, sg1, sg2, sg3, sg4, ss0, ss1, ss2, ss3, ss4, semi):
    semg = [sg0, sg1, sg2, sg3, sg4]
    sems = [ss0, ss1, ss2, ss3, ss4]
    c = lax.axis_index("c")
    s = lax.axis_index("s")
    wid = c * NS + s

    isrc = pltpu.async_copy(src_hbm.at[pl.ds(wid * EPW, EPW)], sidx1, semi)
    idst = pltpu.async_copy(dst_hbm.at[pl.ds(wid * EPW, EPW)], didx1, semi)

    for j in range(K):
        for l in range(DH // 16):
            zr[j, pl.ds(l * 16, 16)] = jnp.zeros((16,), jnp.float32)
    isrc.wait()
    idst.wait()
    _repack_idx(didx1, didx2)

    def dbl(j, _):
        sidx1[pl.ds(j * 16, 16)] = sidx1[pl.ds(j * 16, 16)] * 2
        return ()

    lax.fori_loop(0, EPW // 16, dbl, (), unroll=False)

    for half in range(2):
        if half == 1:
            def bump(j, _):
                sidx1[pl.ds(j * 16, 16)] = sidx1[pl.ds(j * 16, 16)] + 1
                return ()

            lax.fori_loop(0, EPW // 16, bump, (), unroll=False)
        def zinit(j, _):
            ch = j * NS + s

            @pl.when(ch < N // K)
            def _():
                pltpu.sync_copy(zr, acc.at[pl.ds(ch * K, K)])
            return ()

        lax.fori_loop(0, (N // K + NS - 1) // NS, zinit, (), unroll=False)
        plsc.subcore_barrier()

        for b in range(NBUF):
            pltpu.async_copy(hs_hbm.at[sidx1.at[pl.ds(b * K, K)]],
                             rows.at[b], semg[b])

        def block(i0):
            for b in range(NBUF):
                pltpu.make_async_copy(
                    hs_hbm.at[sidx1.at[pl.ds((i0 + b) * K, K)]],
                    rows.at[b], semg[b]).wait()
                pltpu.async_copy(rows.at[b], acc.at[didx2.at[i0 + b]],
                                 sems[b], add=True)
                nxt = i0 + NBUF + b

                @pl.when(nxt < NCHUNK)
                def _():
                    pltpu.make_async_copy(rows.at[b], acc.at[didx2.at[b]],
                                          sems[b]).wait()
                    pltpu.async_copy(hs_hbm.at[sidx1.at[pl.ds(nxt * K, K)]],
                                     rows.at[b], semg[b])

        pl.loop(0, NCHUNK, step=NBUF)(block)
        for b in range(NBUF):
            pltpu.make_async_copy(rows.at[b], acc.at[didx2.at[b]],
                                  sems[b]).wait()
        plsc.subcore_barrier()

        def wb(j, _):
            ch = j * NS + s

            @pl.when(ch < N // K)
            def _():
                pltpu.sync_copy(acc.at[pl.ds(ch * K, K)], rows.at[0])
                pltpu.sync_copy(rows.at[0],
                                out_hbm.at[c, pl.ds(ch * K, K), half])
            return ()

        lax.fori_loop(0, (N // K + NS - 1) // NS, wb, (), unroll=False)
        plsc.subcore_barrier()


@functools.lru_cache(maxsize=None)
def _mp_kernel():
    return pl.kernel(
        _mp_body,
        out_type=jax.ShapeDtypeStruct((NC, N, 2, DH), jnp.float32),
        mesh=_get_mesh(),
        scratch_types=[
            pltpu.VMEM((EPW,), jnp.int32),
            pltpu.VMEM((EPW,), jnp.int32),
            pltpu.VMEM((NCHUNK, K), jnp.int32),
            pltpu.VMEM((NBUF, K, DH), jnp.float32),
            pltpu.VMEM((K, DH), jnp.float32),
            pltpu.VMEM_SHARED((N, DH), jnp.float32),
        ] + [pltpu.SemaphoreType.DMA] * (2 * NBUF + 1),
        compiler_params=pltpu.CompilerParams(use_tc_tiling_on_sc=False),
    )


def _tc_prep_body(x_ref, w1_ref, wres_ref, degt_ref, dis_ref, pre_ref,
                  hs_ref):
    deg = degt_ref[:, 0:1] + degt_ref[:, 1:2] + 1.0
    dis = lax.rsqrt(deg)
    dis_ref[...] = dis
    x = x_ref[...]
    pre_ref[...] = jnp.dot(x, wres_ref[...], preferred_element_type=jnp.float32)
    hs_ref[...] = jnp.dot(x, w1_ref[...], preferred_element_type=jnp.float32) * dis


def _bn_relu(t, g_ref, be_ref):
    mean = jnp.mean(t, axis=0, keepdims=True)
    cen = t - mean
    var = jnp.mean(cen * cen, axis=0, keepdims=True)
    bn = cen * lax.rsqrt(var + 1e-5) * g_ref[...] + be_ref[...]
    return jnp.maximum(bn, 0.0)


def _tc_mid_body(p_ref, hs_ref, dis_ref, pre_ref, b1_ref, g1_ref,
                 be1_ref, w2_ref, h_ref, hs2_ref):
    dis = dis_ref[...]
    t = (p_ref[0] + p_ref[1] + hs_ref[...]) * dis + b1_ref[...]
    h = pre_ref[...] + _bn_relu(t, g1_ref, be1_ref)
    h_ref[...] = h
    hs2_ref[...] = jnp.dot(h, w2_ref[...], preferred_element_type=jnp.float32) * dis


def _tc_final_body(q_ref, hs2_ref, dis_ref, h_ref, b2_ref,
                   g2_ref, be2_ref, out_ref):
    t = (q_ref[0] + q_ref[1] + hs2_ref[...]) * dis_ref[...] + b2_ref[...]
    out_ref[...] = h_ref[...] + _bn_relu(t, g2_ref, be2_ref)


_tc_prep = pl.pallas_call(
    _tc_prep_body,
    out_shape=[
        jax.ShapeDtypeStruct((N, 1), jnp.float32),
        jax.ShapeDtypeStruct((N, D), jnp.float32),
        jax.ShapeDtypeStruct((N, D), jnp.float32),
    ],
)

_tc_mid = pl.pallas_call(
    _tc_mid_body,
    out_shape=[
        jax.ShapeDtypeStruct((N, D), jnp.float32),
        jax.ShapeDtypeStruct((N, D), jnp.float32),
    ],
    compiler_params=pltpu.CompilerParams(vmem_limit_bytes=100 * 1024 * 1024),
)

_tc_final = pl.pallas_call(
    _tc_final_body,
    out_shape=jax.ShapeDtypeStruct((N, D), jnp.float32),
)


def kernel(edge_index, x, W1, b1, g1, be1, W2, b2, g2, be2, Wres):
    src = edge_index[0]
    dst = edge_index[1]
    b1r, g1r, be1r = b1.reshape(1, D), g1.reshape(1, D), be1.reshape(1, D)
    b2r, g2r, be2r = b2.reshape(1, D), g2.reshape(1, D), be2.reshape(1, D)

    degp = _deg_kernel()(dst)
    degt = degp.reshape(NC, N).T
    dis, pre, hs1 = _tc_prep(x, W1, Wres, degt)
    P = _mp_kernel()(src, dst, hs1.reshape(2 * N, DH))
    h, hs2 = _tc_mid(P.reshape(NC, N, D), hs1, dis, pre, b1r, g1r, be1r, W2)
    Q = _mp_kernel()(src, dst, hs2.reshape(2 * N, DH))
    return _tc_final(Q.reshape(NC, N, D), hs2, dis, h, b2r, g2r, be2r)


# speedup vs baseline: 1.3524x; 1.3524x over previous
import functools

import jax
import jax.numpy as jnp
from jax import lax
from jax.experimental import pallas as pl
from jax.experimental.pallas import tpu as pltpu
from jax.experimental.pallas import tpu_sc as plsc

N = 10000
E = 320000
D = 128

NC = 2
NS = 16
NW = NC * NS
EPW = E // NW
K = 80
NCHUNK = EPW // K
NBUF = 5

@functools.lru_cache(maxsize=None)
def _get_mesh():
    return plsc.VectorSubcoreMesh(core_axis_name="c", subcore_axis_name="s",
                                  num_cores=NC, num_subcores=NS)


def _deg_body(dst_hbm, out_hbm, didx2, ones_v, zbuf, dacc, sem, semi):
    c = lax.axis_index("c")
    s = lax.axis_index("s")
    wid = c * NS + s
    idma = pltpu.async_copy(dst_hbm.at[wid], didx2, semi)
    for j in range(2000 // 16):
        zbuf[pl.ds(j * 16, 16)] = jnp.zeros((16,), jnp.float32)
    for j in range(K // 16):
        ones_v[pl.ds(j * 16, 16)] = jnp.full((16,), 1.0, jnp.float32)

    @pl.when(s < 5)
    def _():
        pltpu.sync_copy(zbuf, dacc.at[pl.ds(s * 2000, 2000)])
    idma.wait()
    plsc.subcore_barrier()

    def block(i0):
        sds = [pltpu.async_copy(ones_v, dacc.at[didx2.at[i0 + b]], sem,
                                add=True)
               for b in range(NBUF)]
        for sd in sds:
            sd.wait()

    pl.loop(0, NCHUNK, step=NBUF)(block)
    plsc.subcore_barrier()

    @pl.when(s < 5)
    def _():
        pltpu.sync_copy(dacc.at[pl.ds(s * 2000, 2000)], zbuf)
        pltpu.sync_copy(zbuf, out_hbm.at[pl.ds(c * N + s * 2000, 2000)])


@functools.lru_cache(maxsize=None)
def _deg_kernel():
    return pl.kernel(
        _deg_body,
        out_type=jax.ShapeDtypeStruct((NC * N,), jnp.float32),
        mesh=_get_mesh(),
        scratch_types=[
            pltpu.VMEM((NCHUNK, K), jnp.int32),
            pltpu.VMEM((K,), jnp.float32),
            pltpu.VMEM((2000,), jnp.float32),
            pltpu.VMEM_SHARED((N,), jnp.float32),
            pltpu.SemaphoreType.DMA,
            pltpu.SemaphoreType.DMA,
        ],
        compiler_params=pltpu.CompilerParams(use_tc_tiling_on_sc=False),
    )


DH = D // 2


def _mp_body(src_hbm, dst_hbm, hs_hbm, out_hbm,
             sidx1, didx2, rows, zr, acc,
             sg0, sg1, sg2, sg3, sg4, ss0, ss1, ss2, ss3, ss4, semi):
    semg = [sg0, sg1, sg2, sg3, sg4]
    sems = [ss0, ss1, ss2, ss3, ss4]
    c = lax.axis_index("c")
    s = lax.axis_index("s")
    wid = c * NS + s

    isrc = pltpu.async_copy(src_hbm.at[pl.ds(wid * EPW, EPW)], sidx1, semi)
    idst = pltpu.async_copy(dst_hbm.at[wid], didx2, semi)

    for j in range(K):
        for l in range(DH // 16):
            zr[j, pl.ds(l * 16, 16)] = jnp.zeros((16,), jnp.float32)
    isrc.wait()
    idst.wait()

    def dbl(j, _):
        sidx1[pl.ds(j * 16, 16)] = sidx1[pl.ds(j * 16, 16)] * 2
        return ()

    lax.fori_loop(0, EPW // 16, dbl, (), unroll=False)

    for half in range(2):
        if half == 1:
            def bump(j, _):
                sidx1[pl.ds(j * 16, 16)] = sidx1[pl.ds(j * 16, 16)] + 1
                return ()

            lax.fori_loop(0, EPW // 16, bump, (), unroll=False)
        def zinit(j, _):
            ch = j * NS + s

            @pl.when(ch < N // K)
            def _():
                pltpu.sync_copy(zr, acc.at[pl.ds(ch * K, K)])
            return ()

        lax.fori_loop(0, (N // K + NS - 1) // NS, zinit, (), unroll=False)
        plsc.subcore_barrier()

        for b in range(NBUF):
            pltpu.async_copy(hs_hbm.at[sidx1.at[pl.ds(b * K, K)]],
                             rows.at[b], semg[b])

        def block(i0):
            for b in range(NBUF):
                pltpu.make_async_copy(
                    hs_hbm.at[sidx1.at[pl.ds((i0 + b) * K, K)]],
                    rows.at[b], semg[b]).wait()
                pltpu.async_copy(rows.at[b], acc.at[didx2.at[i0 + b]],
                                 sems[b], add=True)
                nxt = i0 + NBUF + b

                @pl.when(nxt < NCHUNK)
                def _():
                    pltpu.make_async_copy(rows.at[b], acc.at[didx2.at[b]],
                                          sems[b]).wait()
                    pltpu.async_copy(hs_hbm.at[sidx1.at[pl.ds(nxt * K, K)]],
                                     rows.at[b], semg[b])

        pl.loop(0, NCHUNK, step=NBUF)(block)
        for b in range(NBUF):
            pltpu.make_async_copy(rows.at[b], acc.at[didx2.at[b]],
                                  sems[b]).wait()
        plsc.subcore_barrier()

        def wb(j, _):
            ch = j * NS + s

            @pl.when(ch < N // K)
            def _():
                pltpu.sync_copy(acc.at[pl.ds(ch * K, K)], rows.at[0])
                pltpu.sync_copy(rows.at[0],
                                out_hbm.at[half, c, pl.ds(ch * K, K)])
            return ()

        lax.fori_loop(0, (N // K + NS - 1) // NS, wb, (), unroll=False)
        plsc.subcore_barrier()


@functools.lru_cache(maxsize=None)
def _mp_kernel():
    return pl.kernel(
        _mp_body,
        out_type=jax.ShapeDtypeStruct((2, NC, N, DH), jnp.float32),
        mesh=_get_mesh(),
        scratch_types=[
            pltpu.VMEM((EPW,), jnp.int32),
            pltpu.VMEM((NCHUNK, K), jnp.int32),
            pltpu.VMEM((NBUF, K, DH), jnp.float32),
            pltpu.VMEM((K, DH), jnp.float32),
            pltpu.VMEM_SHARED((N, DH), jnp.float32),
        ] + [pltpu.SemaphoreType.DMA] * (2 * NBUF + 1),
        compiler_params=pltpu.CompilerParams(use_tc_tiling_on_sc=False),
    )


def _tc_prep_body(x_ref, w1_ref, wres_ref, degt_ref, dis_ref, pre_ref,
                  hs_ref):
    deg = degt_ref[:, 0:1] + degt_ref[:, 1:2] + 1.0
    dis = lax.rsqrt(deg)
    dis_ref[...] = dis
    x = x_ref[...]
    pre_ref[...] = jnp.dot(x, wres_ref[...], preferred_element_type=jnp.float32)
    hs_ref[...] = jnp.dot(x, w1_ref[...], preferred_element_type=jnp.float32) * dis


def _bn_relu(t, g_ref, be_ref):
    mean = jnp.mean(t, axis=0, keepdims=True)
    cen = t - mean
    var = jnp.mean(cen * cen, axis=0, keepdims=True)
    bn = cen * lax.rsqrt(var + 1e-5) * g_ref[...] + be_ref[...]
    return jnp.maximum(bn, 0.0)


def _gather_sum(p_ref, hs_ref):
    lo = p_ref[0, 0] + p_ref[0, 1]
    hi = p_ref[1, 0] + p_ref[1, 1]
    return jnp.concatenate([lo, hi], axis=1) + hs_ref[...]


def _tc_mid_body(p_ref, hs_ref, dis_ref, pre_ref, b1_ref, g1_ref,
                 be1_ref, w2_ref, h_ref, hs2_ref):
    dis = dis_ref[...]
    t = _gather_sum(p_ref, hs_ref) * dis + b1_ref[...]
    h = pre_ref[...] + _bn_relu(t, g1_ref, be1_ref)
    h_ref[...] = h
    hs2_ref[...] = jnp.dot(h, w2_ref[...], preferred_element_type=jnp.float32) * dis


def _tc_final_body(q_ref, hs2_ref, dis_ref, h_ref, b2_ref,
                   g2_ref, be2_ref, out_ref):
    t = _gather_sum(q_ref, hs2_ref) * dis_ref[...] + b2_ref[...]
    out_ref[...] = h_ref[...] + _bn_relu(t, g2_ref, be2_ref)


_tc_prep = pl.pallas_call(
    _tc_prep_body,
    out_shape=[
        jax.ShapeDtypeStruct((N, 1), jnp.float32),
        jax.ShapeDtypeStruct((N, D), jnp.float32),
        jax.ShapeDtypeStruct((N, D), jnp.float32),
    ],
)

_tc_mid = pl.pallas_call(
    _tc_mid_body,
    out_shape=[
        jax.ShapeDtypeStruct((N, D), jnp.float32),
        jax.ShapeDtypeStruct((N, D), jnp.float32),
    ],
    compiler_params=pltpu.CompilerParams(vmem_limit_bytes=100 * 1024 * 1024),
)

_tc_final = pl.pallas_call(
    _tc_final_body,
    out_shape=jax.ShapeDtypeStruct((N, D), jnp.float32),
)


def kernel(edge_index, x, W1, b1, g1, be1, W2, b2, g2, be2, Wres):
    src = edge_index[0]
    dst = edge_index[1]
    b1r, g1r, be1r = b1.reshape(1, D), g1.reshape(1, D), be1.reshape(1, D)
    b2r, g2r, be2r = b2.reshape(1, D), g2.reshape(1, D), be2.reshape(1, D)

    dst3 = dst.reshape(NW, NCHUNK, K)
    degp = _deg_kernel()(dst3)
    degt = degp.reshape(NC, N).T
    dis, pre, hs1 = _tc_prep(x, W1, Wres, degt)
    P = _mp_kernel()(src, dst3, hs1.reshape(2 * N, DH))
    h, hs2 = _tc_mid(P, hs1, dis, pre, b1r, g1r, be1r, W2)
    Q = _mp_kernel()(src, dst3, hs2.reshape(2 * N, DH))
    return _tc_final(Q, hs2, dis, h, b2r, g2r, be2r)


# speedup vs baseline: 1.3890x; 1.0271x over previous
import functools

import jax
import jax.numpy as jnp
from jax import lax
from jax.experimental import pallas as pl
from jax.experimental.pallas import tpu as pltpu
from jax.experimental.pallas import tpu_sc as plsc

N = 10000
E = 320000
D = 128

NC = 2
NS = 16
NW = NC * NS
EPW = E // NW
K = 80
NCHUNK = EPW // K
NBUF = 5

@functools.lru_cache(maxsize=None)
def _get_mesh():
    return plsc.VectorSubcoreMesh(core_axis_name="c", subcore_axis_name="s",
                                  num_cores=NC, num_subcores=NS)


def _deg_body(dst_hbm, out_hbm, didx2, ones_v, zbuf, dacc, sem, semi):
    c = lax.axis_index("c")
    s = lax.axis_index("s")
    wid = c * NS + s
    idma = pltpu.async_copy(dst_hbm.at[wid], didx2, semi)
    for j in range(2000 // 16):
        zbuf[pl.ds(j * 16, 16)] = jnp.zeros((16,), jnp.float32)
    for j in range(K // 16):
        ones_v[pl.ds(j * 16, 16)] = jnp.full((16,), 1.0, jnp.float32)

    @pl.when(s < 5)
    def _():
        pltpu.sync_copy(zbuf, dacc.at[pl.ds(s * 2000, 2000)])
    idma.wait()
    plsc.subcore_barrier()

    def block(i0):
        sds = [pltpu.async_copy(ones_v, dacc.at[didx2.at[i0 + b]], sem,
                                add=True)
               for b in range(NBUF)]
        for sd in sds:
            sd.wait()

    pl.loop(0, NCHUNK, step=NBUF)(block)
    plsc.subcore_barrier()

    @pl.when(s < 5)
    def _():
        pltpu.sync_copy(dacc.at[pl.ds(s * 2000, 2000)], zbuf)
        pltpu.sync_copy(zbuf, out_hbm.at[pl.ds(c * N + s * 2000, 2000)])


@functools.lru_cache(maxsize=None)
def _deg_kernel():
    return pl.kernel(
        _deg_body,
        out_type=jax.ShapeDtypeStruct((NC * N,), jnp.float32),
        mesh=_get_mesh(),
        scratch_types=[
            pltpu.VMEM((NCHUNK, K), jnp.int32),
            pltpu.VMEM((K,), jnp.float32),
            pltpu.VMEM((2000,), jnp.float32),
            pltpu.VMEM_SHARED((N,), jnp.float32),
            pltpu.SemaphoreType.DMA,
            pltpu.SemaphoreType.DMA,
        ],
        compiler_params=pltpu.CompilerParams(use_tc_tiling_on_sc=False),
    )


DH = D // 2


def _mp_body(src_hbm, dst_hbm, hs_hbm, out_hbm,
             sidx1, didx2, rows, zr, acc,
             sg0, sg1, sg2, sg3, sg4, ss0, ss1, ss2, ss3, ss4, semi):
    semg = [sg0, sg1, sg2, sg3, sg4]
    sems = [ss0, ss1, ss2, ss3, ss4]
    c = lax.axis_index("c")
    s = lax.axis_index("s")
    wid = c * NS + s

    isrc = pltpu.async_copy(src_hbm.at[pl.ds(wid * EPW, EPW)], sidx1, semi)
    idst = pltpu.async_copy(dst_hbm.at[wid], didx2, semi)

    for j in range(K):
        for l in range(DH // 16):
            zr[j, pl.ds(l * 16, 16)] = jnp.zeros((16,), jnp.float32)
    isrc.wait()
    idst.wait()

    def dbl(j, _):
        sidx1[pl.ds(j * 16, 16)] = sidx1[pl.ds(j * 16, 16)] * 2
        return ()

    lax.fori_loop(0, EPW // 16, dbl, (), unroll=False)

    for half in range(2):
        if half == 1:
            def bump(j, _):
                sidx1[pl.ds(j * 16, 16)] = sidx1[pl.ds(j * 16, 16)] + 1
                return ()

            lax.fori_loop(0, EPW // 16, bump, (), unroll=False)
        NZ = (N // K + NS - 1) // NS
        for j in range(NZ):
            ch = j * NS + s

            @pl.when(ch < N // K)
            def _():
                pltpu.async_copy(zr, acc.at[pl.ds(ch * K, K)], semi)
        for j in range(NZ):
            ch = j * NS + s

            @pl.when(ch < N // K)
            def _():
                pltpu.make_async_copy(zr, acc.at[pl.ds(ch * K, K)],
                                      semi).wait()
        plsc.subcore_barrier()

        for b in range(NBUF):
            pltpu.async_copy(hs_hbm.at[sidx1.at[pl.ds(b * K, K)]],
                             rows.at[b], semg[b])

        def block(i0):
            for b in range(NBUF):
                pltpu.make_async_copy(
                    hs_hbm.at[sidx1.at[pl.ds((i0 + b) * K, K)]],
                    rows.at[b], semg[b]).wait()
                pltpu.async_copy(rows.at[b], acc.at[didx2.at[i0 + b]],
                                 sems[b], add=True)
                nxt = i0 + NBUF + b

                @pl.when(nxt < NCHUNK)
                def _():
                    pltpu.make_async_copy(rows.at[b], acc.at[didx2.at[b]],
                                          sems[b]).wait()
                    pltpu.async_copy(hs_hbm.at[sidx1.at[pl.ds(nxt * K, K)]],
                                     rows.at[b], semg[b])

        pl.loop(0, NCHUNK, step=NBUF)(block)
        for b in range(NBUF):
            pltpu.make_async_copy(rows.at[b], acc.at[didx2.at[b]],
                                  sems[b]).wait()
        plsc.subcore_barrier()

        def _wb_hbm(j):
            ch = j * NS + s
            b = j % 2
            return pltpu.make_async_copy(
                rows.at[b], out_hbm.at[half, c, pl.ds(ch * K, K)], sems[b])

        for j in range(NZ):
            ch = j * NS + s
            b = j % 2

            @pl.when(ch < N // K)
            def _():
                if j >= 2:
                    _wb_hbm(j - 2).wait()
                pltpu.async_copy(acc.at[pl.ds(ch * K, K)], rows.at[b],
                                 semg[b])
                pltpu.make_async_copy(acc.at[pl.ds(ch * K, K)], rows.at[b],
                                      semg[b]).wait()
                pltpu.async_copy(rows.at[b],
                                 out_hbm.at[half, c, pl.ds(ch * K, K)],
                                 sems[b])
        for j in range(max(0, NZ - 3), NZ):
            ch = j * NS + s

            @pl.when((ch < N // K) & (ch + 2 * NS >= N // K))
            def _():
                _wb_hbm(j).wait()
        plsc.subcore_barrier()


@functools.lru_cache(maxsize=None)
def _mp_kernel():
    return pl.kernel(
        _mp_body,
        out_type=jax.ShapeDtypeStruct((2, NC, N, DH), jnp.float32),
        mesh=_get_mesh(),
        scratch_types=[
            pltpu.VMEM((EPW,), jnp.int32),
            pltpu.VMEM((NCHUNK, K), jnp.int32),
            pltpu.VMEM((NBUF, K, DH), jnp.float32),
            pltpu.VMEM((K, DH), jnp.float32),
            pltpu.VMEM_SHARED((N, DH), jnp.float32),
        ] + [pltpu.SemaphoreType.DMA] * (2 * NBUF + 1),
        compiler_params=pltpu.CompilerParams(use_tc_tiling_on_sc=False),
    )


def _tc_prep_body(x_ref, w1_ref, wres_ref, degt_ref, dis_ref, pre_ref,
                  hs_ref):
    deg = degt_ref[:, 0:1] + degt_ref[:, 1:2] + 1.0
    dis = lax.rsqrt(deg)
    dis_ref[...] = dis
    x = x_ref[...]
    pre_ref[...] = jnp.dot(x, wres_ref[...], preferred_element_type=jnp.float32)
    hs_ref[...] = jnp.dot(x, w1_ref[...], preferred_element_type=jnp.float32) * dis


def _bn_relu(t, g_ref, be_ref):
    mean = jnp.mean(t, axis=0, keepdims=True)
    cen = t - mean
    var = jnp.mean(cen * cen, axis=0, keepdims=True)
    bn = cen * lax.rsqrt(var + 1e-5) * g_ref[...] + be_ref[...]
    return jnp.maximum(bn, 0.0)


def _gather_sum(p_ref, hs_ref):
    lo = p_ref[0, 0] + p_ref[0, 1]
    hi = p_ref[1, 0] + p_ref[1, 1]
    return jnp.concatenate([lo, hi], axis=1) + hs_ref[...]


def _tc_mid_body(p_ref, hs_ref, dis_ref, pre_ref, b1_ref, g1_ref,
                 be1_ref, w2_ref, h_ref, hs2_ref):
    dis = dis_ref[...]
    t = _gather_sum(p_ref, hs_ref) * dis + b1_ref[...]
    h = pre_ref[...] + _bn_relu(t, g1_ref, be1_ref)
    h_ref[...] = h
    hs2_ref[...] = jnp.dot(h, w2_ref[...], preferred_element_type=jnp.float32) * dis


def _tc_final_body(q_ref, hs2_ref, dis_ref, h_ref, b2_ref,
                   g2_ref, be2_ref, out_ref):
    t = _gather_sum(q_ref, hs2_ref) * dis_ref[...] + b2_ref[...]
    out_ref[...] = h_ref[...] + _bn_relu(t, g2_ref, be2_ref)


_tc_prep = pl.pallas_call(
    _tc_prep_body,
    out_shape=[
        jax.ShapeDtypeStruct((N, 1), jnp.float32),
        jax.ShapeDtypeStruct((N, D), jnp.float32),
        jax.ShapeDtypeStruct((N, D), jnp.float32),
    ],
)

_tc_mid = pl.pallas_call(
    _tc_mid_body,
    out_shape=[
        jax.ShapeDtypeStruct((N, D), jnp.float32),
        jax.ShapeDtypeStruct((N, D), jnp.float32),
    ],
    compiler_params=pltpu.CompilerParams(vmem_limit_bytes=100 * 1024 * 1024),
)

_tc_final = pl.pallas_call(
    _tc_final_body,
    out_shape=jax.ShapeDtypeStruct((N, D), jnp.float32),
)


def kernel(edge_index, x, W1, b1, g1, be1, W2, b2, g2, be2, Wres):
    src = edge_index[0]
    dst = edge_index[1]
    b1r, g1r, be1r = b1.reshape(1, D), g1.reshape(1, D), be1.reshape(1, D)
    b2r, g2r, be2r = b2.reshape(1, D), g2.reshape(1, D), be2.reshape(1, D)

    dst3 = dst.reshape(NW, NCHUNK, K)
    degp = _deg_kernel()(dst3)
    degt = degp.reshape(NC, N).T
    dis, pre, hs1 = _tc_prep(x, W1, Wres, degt)
    P = _mp_kernel()(src, dst3, hs1.reshape(2 * N, DH))
    h, hs2 = _tc_mid(P, hs1, dis, pre, b1r, g1r, be1r, W2)
    Q = _mp_kernel()(src, dst3, hs2.reshape(2 * N, DH))
    return _tc_final(Q, hs2, dis, h, b2r, g2r, be2r)
